# baseline (speedup 1.0000x reference)
import functools

import jax
import jax.numpy as jnp
from jax import lax
from jax.experimental import pallas as pl
from jax.experimental.pallas import tpu as pltpu
from jax.experimental.pallas import tpu_sc as plsc

B = 2
S = 2048
T = B * S
D = 1024
F = 2048
E = 8
K = 2
A = T * K

TM = 256
P = A + E * TM
NT = P // TM
FCH = 512
NFC = F // FCH

NC = 2
NS = 16
NW = NC * NS


def _linear_body(x_ref, wl_ref, b_ref, wg_ref, flat_ref, log_ref):
    acc = jnp.dot(x_ref[...], wl_ref[...],
                  preferred_element_type=jnp.float32) + b_ref[...]
    flat_ref[...] = acc
    log_ref[...] = jnp.dot(acc, wg_ref[...],
                           preferred_element_type=jnp.float32)


def _linear(x2, W_lin, b_lin, Wg1):
    return pl.pallas_call(
        _linear_body,
        grid=(T // TM,),
        in_specs=[
            pl.BlockSpec((TM, D), lambda i: (i, 0)),
            pl.BlockSpec((D, D), lambda i: (0, 0)),
            pl.BlockSpec((1, D), lambda i: (0, 0)),
            pl.BlockSpec((D, E), lambda i: (0, 0)),
        ],
        out_specs=[
            pl.BlockSpec((TM, D), lambda i: (i, 0)),
            pl.BlockSpec((TM, E), lambda i: (i, 0)),
        ],
        out_shape=[
            jax.ShapeDtypeStruct((T, D), jnp.float32),
            jax.ShapeDtypeStruct((T, E), jnp.float32),
        ],
    )(x2, W_lin, b_lin.reshape(1, D), Wg1)


def _router_body(m_ref, wg_ref, log_ref):
    log_ref[...] = jnp.dot(m_ref[...], wg_ref[...],
                           preferred_element_type=jnp.float32)


def _router(m1, Wg2):
    return pl.pallas_call(
        _router_body,
        grid=(T // TM,),
        in_specs=[
            pl.BlockSpec((TM, D), lambda i: (i, 0)),
            pl.BlockSpec((D, E), lambda i: (0, 0)),
        ],
        out_specs=pl.BlockSpec((TM, E), lambda i: (i, 0)),
        out_shape=jax.ShapeDtypeStruct((T, E), jnp.float32),
    )(m1, Wg2)


def _route(logits):
    probs = jax.nn.softmax(logits, axis=-1)
    topv, topi = lax.top_k(probs, K)
    gates = topv / jnp.sum(topv, axis=-1, keepdims=True)

    e = topi.reshape(-1).astype(jnp.int32)
    g = gates.reshape(-1)
    oh = (e[:, None] == jnp.arange(E, dtype=jnp.int32)).astype(jnp.int32)
    cum = jnp.cumsum(oh, axis=0)
    rank = cum[jnp.arange(A), e] - 1
    counts = cum[-1]
    padded = ((counts + TM - 1) // TM) * TM
    ends = jnp.cumsum(padded)
    gstart = ends - padded
    dest = (gstart[e] + rank).astype(jnp.int32)

    tok = jnp.arange(A, dtype=jnp.int32) // K
    row_index = jnp.zeros((P,), jnp.int32).at[dest].set(tok)
    gate_s = jnp.zeros((P,), jnp.float32).at[dest].set(g)
    tile_expert = jnp.searchsorted(
        ends, jnp.arange(NT, dtype=jnp.int32) * TM, side='right')
    tile_expert = jnp.minimum(tile_expert, E - 1).astype(jnp.int32)
    inv = dest.reshape(T, K)
    return row_index, gate_s.reshape(P, 1), tile_expert, inv[:, 0], inv[:, 1]


_GCH = 64
_GROWS = P // NW


@functools.cache
def _build_sc_gather():
    mesh = plsc.VectorSubcoreMesh(core_axis_name="c", subcore_axis_name="s")

    @functools.partial(
        pl.kernel,
        mesh=mesh,
        out_type=jax.ShapeDtypeStruct((P, D), jnp.float32),
        scratch_types=[
            pltpu.VMEM((_GCH,), jnp.int32),
            pltpu.VMEM((_GCH, D), jnp.float32),
            pltpu.SemaphoreType.DMA,
        ],
    )
    def gather_k(src_hbm, idx_hbm, out_hbm, idx_v, rows_v, sem):
        wid = lax.axis_index("s") * NC + lax.axis_index("c")
        base = wid * _GROWS
        for c in range(_GROWS // _GCH):
            off = base + c * _GCH
            pltpu.sync_copy(idx_hbm.at[pl.ds(off, _GCH)], idx_v)
            pltpu.async_copy(src_hbm.at[idx_v], rows_v, sem).wait()
            pltpu.sync_copy(rows_v, out_hbm.at[pl.ds(off, _GCH)])

    return gather_k


def _sc_gather(src, idx):
    return _build_sc_gather()(src, idx)


_CCH = 32
_CROWS = T // NW


@functools.cache
def _build_sc_combine():
    mesh = plsc.VectorSubcoreMesh(core_axis_name="c", subcore_axis_name="s")

    @functools.partial(
        pl.kernel,
        mesh=mesh,
        out_type=jax.ShapeDtypeStruct((T, D), jnp.float32),
        scratch_types=[
            pltpu.VMEM((_CCH,), jnp.int32),
            pltpu.VMEM((_CCH,), jnp.int32),
            pltpu.VMEM((_CCH, D), jnp.float32),
            pltpu.VMEM((_CCH, D), jnp.float32),
            pltpu.SemaphoreType.DMA,
        ],
    )
    def combine_k(yg_hbm, inv0_hbm, inv1_hbm, out_hbm, i0v, i1v, b0, b1, sem):
        wid = lax.axis_index("s") * NC + lax.axis_index("c")
        base = wid * _CROWS
        for c in range(_CROWS // _CCH):
            off = base + c * _CCH
            pltpu.sync_copy(inv0_hbm.at[pl.ds(off, _CCH)], i0v)
            pltpu.sync_copy(inv1_hbm.at[pl.ds(off, _CCH)], i1v)
            pltpu.async_copy(yg_hbm.at[i0v], b0, sem).wait()
            pltpu.async_copy(yg_hbm.at[i1v], b1, sem).wait()

            def add_row(r, carry):
                def add_col(cc, carry2):
                    sl = pl.ds(cc * 16, 16)
                    b0[r, sl] = b0[r, sl] + b1[r, sl]
                    return carry2
                return lax.fori_loop(0, D // 16, add_col, carry)

            lax.fori_loop(0, _CCH, add_row, 0)
            pltpu.sync_copy(b0, out_hbm.at[pl.ds(off, _CCH)])

    return combine_k


def _sc_combine(yg, inv0, inv1):
    return _build_sc_combine()(yg, inv0, inv1)


def _moe_body(te_ref, xg_ref, win_ref, wout_ref, g_ref, yg_ref, acc_ref):
    j = pl.program_id(1)
    h = jax.nn.gelu(jnp.dot(xg_ref[...], win_ref[0],
                            preferred_element_type=jnp.float32))
    prod = jnp.dot(h, wout_ref[0], preferred_element_type=jnp.float32)

    @pl.when(j == 0)
    def _():
        acc_ref[...] = prod

    @pl.when(j > 0)
    def _():
        acc_ref[...] += prod

    @pl.when(j == NFC - 1)
    def _():
        yg_ref[...] = acc_ref[...] * g_ref[...]


def _moe(xg, W_in, W_out, gates2d, tile_expert):
    grid_spec = pltpu.PrefetchScalarGridSpec(
        num_scalar_prefetch=1,
        grid=(NT, NFC),
        in_specs=[
            pl.BlockSpec((TM, D), lambda i, j, te: (i, 0)),
            pl.BlockSpec((1, D, FCH), lambda i, j, te: (te[i], 0, j)),
            pl.BlockSpec((1, FCH, D), lambda i, j, te: (te[i], j, 0)),
            pl.BlockSpec((TM, 1), lambda i, j, te: (i, 0)),
        ],
        out_specs=pl.BlockSpec((TM, D), lambda i, j, te: (i, 0)),
        scratch_shapes=[pltpu.VMEM((TM, D), jnp.float32)],
    )
    return pl.pallas_call(
        _moe_body,
        grid_spec=grid_spec,
        out_shape=jax.ShapeDtypeStruct((P, D), jnp.float32),
    )(tile_expert, xg, W_in, W_out, gates2d)


def _final_body(y_ref, flat_ref, m2_ref, out_ref, acc_ref):
    i = pl.program_id(0)

    @pl.when(i == 0)
    def _():
        acc_ref[...] = jnp.zeros_like(acc_ref)

    rows = flat_ref[...] + m2_ref[...]
    part = jnp.sum(rows, axis=0, keepdims=True)
    b = i // (S // TM)
    rowi = lax.broadcasted_iota(jnp.int32, (8, D), 0)
    acc_ref[...] += jnp.where(rowi == b, part, 0.0)

    @pl.when(i == T // TM - 1)
    def _():
        sent = acc_ref[...] / jnp.float32(S)
        mx = jnp.max(sent, axis=1, keepdims=True)
        z = sent - mx
        lse = jnp.log(jnp.sum(jnp.exp(z), axis=1, keepdims=True))
        logp = z - lse
        coli = lax.broadcasted_iota(jnp.int32, (8, D), 1)
        rowj = lax.broadcasted_iota(jnp.int32, (8, D), 0)
        sel = (((rowj == 0) & (coli == y_ref[0]))
               | ((rowj == 1) & (coli == y_ref[1])))
        loss = -jnp.sum(jnp.where(sel, logp, 0.0)) / jnp.float32(B)
        out_ref[...] = jnp.full((8, 128), loss, jnp.float32)


def _final(flat, m2, y):
    grid_spec = pltpu.PrefetchScalarGridSpec(
        num_scalar_prefetch=1,
        grid=(T // TM,),
        in_specs=[
            pl.BlockSpec((TM, D), lambda i, y_ref: (i, 0)),
            pl.BlockSpec((TM, D), lambda i, y_ref: (i, 0)),
        ],
        out_specs=pl.BlockSpec((8, 128), lambda i, y_ref: (0, 0)),
        scratch_shapes=[pltpu.VMEM((8, D), jnp.float32)],
    )
    return pl.pallas_call(
        _final_body,
        grid_spec=grid_spec,
        out_shape=jax.ShapeDtypeStruct((8, 128), jnp.float32),
    )(y, flat, m2)


def kernel(x, y, W_lin, b_lin, Wg1, W1_in, W1_out, Wg2, W2_in, W2_out):
    x2 = x.reshape(T, D)
    flat, logits1 = _linear(x2, W_lin, b_lin, Wg1)

    row1, g1, te1, i10, i11 = _route(logits1)
    xg1 = _sc_gather(flat, row1)
    yg1 = _moe(xg1, W1_in, W1_out, g1, te1)
    m1 = _sc_combine(yg1, i10, i11)

    logits2 = _router(m1, Wg2)
    row2, g2, te2, i20, i21 = _route(logits2)
    xg2 = _sc_gather(m1, row2)
    yg2 = _moe(xg2, W2_in, W2_out, g2, te2)
    m2 = _sc_combine(yg2, i20, i21)

    loss = _final(flat, m2, y.astype(jnp.int32))
    return loss[0, 0]
